# baseline (speedup 1.0000x reference)
import functools

import jax
import jax.numpy as jnp
from jax import lax
from jax.experimental import pallas as pl
from jax.experimental.pallas import tpu as pltpu
from jax.experimental.pallas import tpu_sc as plsc

_CPAD = 32


def _seg_body(c, h, w, f_ref, l_ref, out_ref, acc):
    step = pl.program_id(0) * pl.num_programs(1) + pl.program_id(1)
    nsteps = pl.num_programs(0) * pl.num_programs(1)

    @pl.when(step == 0)
    def _init():
        acc[...] = jnp.zeros_like(acc)

    f = f_ref[0].reshape(c, h * w)
    labs = l_ref[0].reshape(1, h * w)

    klass = lax.broadcasted_iota(jnp.int32, (_CPAD, h * w), 0)
    m = (klass == labs).astype(jnp.float32)

    rowsq = jnp.sum(f * f, axis=0, keepdims=True)
    g = jnp.concatenate([f, rowsq], axis=0)
    acc[...] += lax.dot_general(
        g, m, (((1,), (1,)), ((), ())),
        preferred_element_type=jnp.float32)

    @pl.when(step == nsteps - 1)
    def _finish():
        out_ref[...] = acc[...]


_HSPLIT = 4


def _tc_segment_sums(features, labs4d):
    b, c, h, w = features.shape
    hb = h // _HSPLIT
    return pl.pallas_call(
        functools.partial(_seg_body, c, hb, w),
        grid=(b, _HSPLIT),
        in_specs=[
            pl.BlockSpec((1, c, hb, w), lambda i, j: (i, 0, j, 0)),
            pl.BlockSpec((1, 1, hb, w), lambda i, j: (i, 0, j, 0)),
        ],
        out_specs=pl.BlockSpec((c + 1, _CPAD), lambda i, j: (0, 0)),
        out_shape=jax.ShapeDtypeStruct((c + 1, _CPAD), jnp.float32),
        scratch_shapes=[
            pltpu.VMEM((c + 1, _CPAD), jnp.float32),
        ],
    )(features, labs4d)


_NC = 2
_NS = 16
_NW = _NC * _NS
_LANES = 16


def _sc_counts(labs_flat):
    n = labs_flat.shape[0]
    per_w = n // _NW
    mesh = plsc.VectorSubcoreMesh(core_axis_name="c", subcore_axis_name="s")

    @functools.partial(
        pl.kernel,
        mesh=mesh,
        out_type=jax.ShapeDtypeStruct((_NW, 128), jnp.float32),
        scratch_types=[
            pltpu.VMEM((per_w,), jnp.int32),
            pltpu.VMEM((128,), jnp.float32),
        ],
        compiler_params=pltpu.CompilerParams(needs_layout_passes=False),
    )
    def hist(labels_hbm, out_hbm, lab_v, acc_v):
        wid = lax.axis_index("s") * _NC + lax.axis_index("c")
        base = wid * per_w
        pltpu.sync_copy(labels_hbm.at[pl.ds(base, per_w)], lab_v)
        for z in range(8):
            acc_v[pl.ds(z * _LANES, _LANES)] = jnp.zeros((_LANES,), jnp.float32)
        ones = jnp.ones((_LANES,), jnp.float32)

        def body(i, carry):
            lab = lab_v[pl.ds(i * _LANES, _LANES)]
            plsc.addupdate_scatter(acc_v, [lab], ones)
            return carry

        lax.fori_loop(0, per_w // _LANES, body, 0)
        pltpu.sync_copy(acc_v, out_hbm.at[wid])

    return hist(labs_flat)


def kernel(features, labels, prototypes):
    b, c, h, w = features.shape
    ncls = prototypes.shape[0]

    labs4d = labels.astype(jnp.int32).reshape(b, 1, h, w)
    labs_flat = labels.astype(jnp.int32).reshape(b * h * w)

    part = _sc_counts(labs_flat)
    sq = _tc_segment_sums(features, labs4d)

    n = jnp.sum(part, axis=0)[:ncls]
    s = sq[:c, :ncls]
    q = sq[c, :ncls]

    ps = jnp.sum(prototypes.T * s, axis=0)
    pp = jnp.sum(prototypes * prototypes, axis=1)
    present = n > 0.0
    denom = jnp.where(present, n, 1.0) * jnp.float32(c)
    term = jnp.where(present, (q + n * pp - 2.0 * ps) / denom, 0.0)
    return jnp.sum(term) / jnp.sum(present.astype(jnp.float32))


# speedup vs baseline: 1.2831x; 1.2831x over previous
import functools

import jax
import jax.numpy as jnp
from jax import lax
from jax.experimental import pallas as pl
from jax.experimental.pallas import tpu as pltpu
from jax.experimental.pallas import tpu_sc as plsc

_CPAD = 32


def _seg_body(c, h, w, f_ref, l_ref, out_ref, acc):
    step = pl.program_id(0) * pl.num_programs(1) + pl.program_id(1)
    nsteps = pl.num_programs(0) * pl.num_programs(1)

    @pl.when(step == 0)
    def _init():
        acc[...] = jnp.zeros_like(acc)

    f = jnp.concatenate(
        [f_ref[i].reshape(c, h * w) for i in range(f_ref.shape[0])],
        axis=1)
    labs = jnp.concatenate(
        [l_ref[i].reshape(1, h * w) for i in range(l_ref.shape[0])],
        axis=1)

    klass = lax.broadcasted_iota(jnp.int32, (_CPAD, labs.shape[1]), 0)
    m = (klass == labs).astype(jnp.float32)

    rowsq = jnp.sum(f * f, axis=0, keepdims=True)
    g = jnp.concatenate([f, rowsq], axis=0)
    acc[...] += lax.dot_general(
        g, m, (((1,), (1,)), ((), ())),
        preferred_element_type=jnp.float32)

    @pl.when(step == nsteps - 1)
    def _finish():
        out_ref[...] = acc[...]


_BB = 2


def _tc_segment_sums(features, labs4d):
    b, c, h, w = features.shape
    return pl.pallas_call(
        functools.partial(_seg_body, c, h, w),
        grid=(b // _BB, 1),
        in_specs=[
            pl.BlockSpec((_BB, c, h, w), lambda i, j: (i, 0, 0, 0)),
            pl.BlockSpec((_BB, 1, h, w), lambda i, j: (i, 0, 0, 0)),
        ],
        out_specs=pl.BlockSpec((c + 1, _CPAD), lambda i, j: (0, 0)),
        out_shape=jax.ShapeDtypeStruct((c + 1, _CPAD), jnp.float32),
        scratch_shapes=[
            pltpu.VMEM((c + 1, _CPAD), jnp.float32),
        ],
    )(features, labs4d)


_NC = 2
_NS = 16
_NW = _NC * _NS
_LANES = 16


def _sc_counts(labs_flat):
    n = labs_flat.shape[0]
    per_w = n // _NW
    mesh = plsc.VectorSubcoreMesh(core_axis_name="c", subcore_axis_name="s")

    @functools.partial(
        pl.kernel,
        mesh=mesh,
        out_type=jax.ShapeDtypeStruct((_NW, 128), jnp.float32),
        scratch_types=[
            pltpu.VMEM((per_w,), jnp.int32),
            pltpu.VMEM((128,), jnp.float32),
        ],
        compiler_params=pltpu.CompilerParams(needs_layout_passes=False),
    )
    def hist(labels_hbm, out_hbm, lab_v, acc_v):
        wid = lax.axis_index("s") * _NC + lax.axis_index("c")
        base = wid * per_w
        pltpu.sync_copy(labels_hbm.at[pl.ds(base, per_w)], lab_v)
        for z in range(8):
            acc_v[pl.ds(z * _LANES, _LANES)] = jnp.zeros((_LANES,), jnp.float32)
        ones = jnp.ones((_LANES,), jnp.float32)

        def body(i, carry):
            lab = lab_v[pl.ds(i * _LANES, _LANES)]
            plsc.addupdate_scatter(acc_v, [lab], ones)
            return carry

        lax.fori_loop(0, per_w // _LANES, body, 0)
        pltpu.sync_copy(acc_v, out_hbm.at[wid])

    return hist(labs_flat)


def kernel(features, labels, prototypes):
    b, c, h, w = features.shape
    ncls = prototypes.shape[0]

    labs4d = labels.astype(jnp.int32).reshape(b, 1, h, w)
    labs_flat = labels.astype(jnp.int32).reshape(b * h * w)

    part = _sc_counts(labs_flat)
    sq = _tc_segment_sums(features, labs4d)

    n = jnp.sum(part, axis=0)[:ncls]
    s = sq[:c, :ncls]
    q = sq[c, :ncls]

    ps = jnp.sum(prototypes.T * s, axis=0)
    pp = jnp.sum(prototypes * prototypes, axis=1)
    present = n > 0.0
    denom = jnp.where(present, n, 1.0) * jnp.float32(c)
    term = jnp.where(present, (q + n * pp - 2.0 * ps) / denom, 0.0)
    return jnp.sum(term) / jnp.sum(present.astype(jnp.float32))


# speedup vs baseline: 2.0961x; 1.6337x over previous
import functools

import jax
import jax.numpy as jnp
from jax import lax
from jax.experimental import pallas as pl
from jax.experimental.pallas import tpu as pltpu

_CPAD = 32


def _loss_body(c, h, w, f_ref, l_ref, pt_ref, out_ref, acc):
    step = pl.program_id(0)
    nsteps = pl.num_programs(0)

    @pl.when(step == 0)
    def _init():
        acc[...] = jnp.zeros_like(acc)

    f = f_ref[0].reshape(c, h * w)
    labs = l_ref[0].reshape(1, h * w)

    klass = lax.broadcasted_iota(jnp.int32, (_CPAD, h * w), 0)
    m = (klass == labs).astype(jnp.bfloat16)

    rowsq = jnp.sum(f * f, axis=0, keepdims=True)
    g = jnp.concatenate(
        [f, rowsq, jnp.ones_like(rowsq)], axis=0)
    acc[...] += lax.dot_general(
        g.astype(jnp.bfloat16), m, (((1,), (1,)), ((), ())),
        preferred_element_type=jnp.float32)

    @pl.when(step == nsteps - 1)
    def _finish():
        s = acc[0:c, :]
        q = acc[c:c + 1, :]
        n = acc[c + 1:c + 2, :]
        pt = pt_ref[...]
        ps = jnp.sum(pt * s, axis=0, keepdims=True)
        pp = jnp.sum(pt * pt, axis=0, keepdims=True)
        present = n > 0.0
        denom = jnp.where(present, n, 1.0) * jnp.float32(c)
        term = jnp.where(present, (q + n * pp - 2.0 * ps) / denom, 0.0)
        loss = jnp.sum(term) / jnp.sum(present.astype(jnp.float32))
        out_ref[0, 0] = loss


def kernel(features, labels, prototypes):
    b, c, h, w = features.shape
    ncls = prototypes.shape[0]

    labs = labels.astype(jnp.int32).reshape(b, 1, h, w)
    pt = jnp.zeros((c, _CPAD), jnp.float32).at[:, :ncls].set(prototypes.T)

    out = pl.pallas_call(
        functools.partial(_loss_body, c, h, w),
        grid=(b,),
        in_specs=[
            pl.BlockSpec((1, c, h, w), lambda i: (i, 0, 0, 0)),
            pl.BlockSpec((1, 1, h, w), lambda i: (i, 0, 0, 0)),
            pl.BlockSpec((c, _CPAD), lambda i: (0, 0)),
        ],
        out_specs=pl.BlockSpec(memory_space=pltpu.SMEM),
        out_shape=jax.ShapeDtypeStruct((1, 1), jnp.float32),
        scratch_shapes=[
            pltpu.VMEM((c + 2, _CPAD), jnp.float32),
        ],
    )(features, labs, pt)
    return out.reshape(())
